# Initial kernel scaffold; baseline (speedup 1.0000x reference)
import jax
import jax.numpy as jnp
from jax.experimental import pallas as pl


def kernel(preds, targets):
    raise NotImplementedError("write your pallas kernel here")


# speedup vs baseline: 1.1642x; 1.1642x over previous
import functools

import jax
import jax.numpy as jnp
from jax import lax
from jax.experimental import pallas as pl
from jax.experimental.pallas import tpu as pltpu
from jax.experimental.pallas import tpu_sc as plsc


def _make_sc_gather(B, V):
    info = plsc.get_sparse_core_info()
    NC, NS, L = info.num_cores, info.num_subcores, info.num_lanes
    NW = NC * NS
    BPW = B // NW
    assert B % (8 * NW) == 0

    mesh = plsc.VectorSubcoreMesh(core_axis_name="c", subcore_axis_name="s")

    @functools.partial(
        pl.kernel,
        mesh=mesh,
        out_type=jax.ShapeDtypeStruct((B,), jnp.float32),
        scratch_types=[
            pltpu.VMEM((BPW,), jnp.int32),
            pltpu.VMEM((BPW,), jnp.float32),
            pltpu.SemaphoreType.DMA,
        ],
    )
    def sc_gather(preds_flat_hbm, tgt_hbm, out_hbm, idx_v, vals_v, sem):
        wid = lax.axis_index("s") * NC + lax.axis_index("c")
        base = wid * BPW
        pltpu.sync_copy(tgt_hbm.at[pl.ds(base, BPW)], idx_v)
        for cc in range(BPW // L):
            t = idx_v[pl.ds(cc * L, L)]
            rows = (base + cc * L) + lax.iota(jnp.int32, L)
            idx_v[pl.ds(cc * L, L)] = rows * V + t
        pltpu.async_copy(preds_flat_hbm.at[idx_v], vals_v, sem).wait()
        pltpu.sync_copy(vals_v, out_hbm.at[pl.ds(base, BPW)])

    return sc_gather


def _count_body(B, V, BN, C, tval_ref, tgt_ref, preds_ref, out_ref, count_ref):
    c = pl.program_id(0)

    @pl.when(c == 0)
    def _init():
        count_ref[...] = jnp.zeros_like(count_ref)

    x = preds_ref[...]
    tv = tval_ref[...]
    ti = tgt_ref[...]
    col = c * BN + lax.broadcasted_iota(jnp.int32, (B, BN), 1)
    better = (x > tv) | ((x == tv) & (col < ti))
    better &= col < V
    count_ref[...] += jnp.sum(better.astype(jnp.int32), axis=1, keepdims=True)

    @pl.when(c == C - 1)
    def _fin():
        rank = count_ref[...]
        c1 = jnp.sum((rank < 1).astype(jnp.float32))
        c5 = jnp.sum((rank < 5).astype(jnp.float32))
        out_ref[0] = (1.0 - c1 / B) * 100.0
        out_ref[1] = (1.0 - c5 / B) * 100.0


def _make_tc_count(B, V, BN=2048):
    C = pl.cdiv(V, BN)
    return pl.pallas_call(
        functools.partial(_count_body, B, V, BN, C),
        grid=(C,),
        in_specs=[
            pl.BlockSpec((B, 1), lambda c: (0, 0)),
            pl.BlockSpec((B, 1), lambda c: (0, 0)),
            pl.BlockSpec((B, BN), lambda c: (0, c)),
        ],
        out_specs=pl.BlockSpec(memory_space=pltpu.SMEM),
        out_shape=jax.ShapeDtypeStruct((2,), jnp.float32),
        scratch_shapes=[pltpu.VMEM((B, 1), jnp.int32)],
        compiler_params=pltpu.CompilerParams(
            dimension_semantics=("arbitrary",),
        ),
    )


def kernel(preds, targets):
    B, V = preds.shape
    targets = targets.astype(jnp.int32)
    tvals = _make_sc_gather(B, V)(preds.reshape(-1), targets)
    errs = _make_tc_count(B, V)(
        tvals.reshape(B, 1), targets.reshape(B, 1), preds
    )
    return (errs[0], errs[1])


# speedup vs baseline: 2.2182x; 1.9053x over previous
import functools

import jax
import jax.numpy as jnp
from jax import lax
from jax.experimental import pallas as pl
from jax.experimental.pallas import tpu as pltpu
from jax.experimental.pallas import tpu_sc as plsc


_SPECS_PER_STEP = 64


def _panel_body(*refs):
    in_refs = refs[1:1 + _SPECS_PER_STEP]
    out_ref = refs[1 + _SPECS_PER_STEP]
    sub = lax.broadcasted_iota(jnp.int32, (_SPECS_PER_STEP, 128), 0)
    acc = jnp.zeros((_SPECS_PER_STEP, 128), jnp.float32)
    for j in range(_SPECS_PER_STEP):
        row = in_refs[j][j % 8:j % 8 + 1, :]
        acc = jnp.where(sub == j, row, acc)
    out_ref[...] = acc


def _make_panel_stage(B, V):
    G = B // _SPECS_PER_STEP

    def mk_index_map(j):
        def im(g, tref):
            rg = g * (_SPECS_PER_STEP // 8) + j // 8
            r = rg * 8 + (j % 8)
            return (rg, tref[r] // 128)
        return im

    grid_spec = pltpu.PrefetchScalarGridSpec(
        num_scalar_prefetch=1,
        grid=(G,),
        in_specs=[pl.BlockSpec((8, 128), mk_index_map(j))
                  for j in range(_SPECS_PER_STEP)],
        out_specs=pl.BlockSpec((_SPECS_PER_STEP, 128), lambda g, tref: (g, 0)),
    )
    call = pl.pallas_call(
        _panel_body,
        grid_spec=grid_spec,
        out_shape=jax.ShapeDtypeStruct((B, 128), jnp.float32),
    )
    return lambda targets, preds: call(targets, *([preds] * _SPECS_PER_STEP))


def _make_sc_gather(B, V):
    info = plsc.get_sparse_core_info()
    NC, NS, L = info.num_cores, info.num_subcores, info.num_lanes
    NW = NC * NS
    BPW = B // NW
    assert B % (8 * NW) == 0

    mesh = plsc.VectorSubcoreMesh(core_axis_name="c", subcore_axis_name="s")

    @functools.partial(
        pl.kernel,
        mesh=mesh,
        out_type=jax.ShapeDtypeStruct((B,), jnp.float32),
        scratch_types=[
            pltpu.VMEM((BPW,), jnp.int32),
            pltpu.VMEM((BPW,), jnp.float32),
            pltpu.SemaphoreType.DMA,
        ],
    )
    def sc_gather(panels_flat_hbm, tgt_hbm, out_hbm, idx_v, vals_v, sem):
        wid = lax.axis_index("s") * NC + lax.axis_index("c")
        base = wid * BPW
        pltpu.sync_copy(tgt_hbm.at[pl.ds(base, BPW)], idx_v)
        for cc in range(BPW // L):
            t = idx_v[pl.ds(cc * L, L)]
            rows = (base + cc * L) + lax.iota(jnp.int32, L)
            idx_v[pl.ds(cc * L, L)] = rows * 128 + (t & 127)
        pltpu.async_copy(panels_flat_hbm.at[idx_v], vals_v, sem).wait()
        pltpu.sync_copy(vals_v, out_hbm.at[pl.ds(base, BPW)])

    return sc_gather


def _count_body(B, V, BN, C, tval_ref, tgt_ref, preds_ref, out_ref, count_ref):
    c = pl.program_id(0)

    @pl.when(c == 0)
    def _init():
        count_ref[...] = jnp.zeros_like(count_ref)

    x = preds_ref[...]
    tv = tval_ref[...]
    ti = tgt_ref[...]
    col = c * BN + lax.broadcasted_iota(jnp.int32, (B, BN), 1)
    better = (x > tv) | ((x == tv) & (col < ti))
    better &= col < V
    count_ref[...] += jnp.sum(better.astype(jnp.int32), axis=1, keepdims=True)

    @pl.when(c == C - 1)
    def _fin():
        rank = count_ref[...]
        c1 = jnp.sum((rank < 1).astype(jnp.float32))
        c5 = jnp.sum((rank < 5).astype(jnp.float32))
        out_ref[0] = (1.0 - c1 / B) * 100.0
        out_ref[1] = (1.0 - c5 / B) * 100.0


def _make_tc_count(B, V, BN=2048):
    C = pl.cdiv(V, BN)
    return pl.pallas_call(
        functools.partial(_count_body, B, V, BN, C),
        grid=(C,),
        in_specs=[
            pl.BlockSpec((B, 1), lambda c: (0, 0)),
            pl.BlockSpec((B, 1), lambda c: (0, 0)),
            pl.BlockSpec((B, BN), lambda c: (0, c)),
        ],
        out_specs=pl.BlockSpec(memory_space=pltpu.SMEM),
        out_shape=jax.ShapeDtypeStruct((2,), jnp.float32),
        scratch_shapes=[pltpu.VMEM((B, 1), jnp.int32)],
        compiler_params=pltpu.CompilerParams(
            dimension_semantics=("arbitrary",),
        ),
    )


def kernel(preds, targets):
    B, V = preds.shape
    targets = targets.astype(jnp.int32)
    panels = _make_panel_stage(B, V)(targets, preds)
    tvals = _make_sc_gather(B, V)(panels.reshape(-1), targets)
    errs = _make_tc_count(B, V)(
        tvals.reshape(B, 1), targets.reshape(B, 1), preds
    )
    return (errs[0], errs[1])


# speedup vs baseline: 6.9398x; 3.1286x over previous
import functools

import jax
import jax.numpy as jnp
from jax import lax
from jax.experimental import pallas as pl
from jax.experimental.pallas import tpu as pltpu
from jax.experimental.pallas import tpu_sc as plsc


_LANES = 128


def _strip_body(*refs):
    in_refs = refs[1:1 + _LANES]
    out_ref = refs[1 + _LANES]
    lane = lax.broadcasted_iota(jnp.int32, (8, _LANES), 1)
    acc = jnp.zeros((8, _LANES), jnp.float32)
    for j in range(_LANES):
        acc = jnp.where(lane == j, in_refs[j][...], acc)
    out_ref[...] = acc


def _make_strip_stage(B, V):
    G = B // _LANES

    def mk_index_map(j):
        def im(g, tref):
            return (tref[g * _LANES + j] // 8, g)
        return im

    grid_spec = pltpu.PrefetchScalarGridSpec(
        num_scalar_prefetch=1,
        grid=(G,),
        in_specs=[pl.BlockSpec((8, _LANES), mk_index_map(j))
                  for j in range(_LANES)],
        out_specs=pl.BlockSpec((8, _LANES), lambda g, tref: (0, g)),
    )
    call = pl.pallas_call(
        _strip_body,
        grid_spec=grid_spec,
        out_shape=jax.ShapeDtypeStruct((8, B), jnp.float32),
    )
    return lambda targets, predsT: call(targets, *([predsT] * _LANES))


def _make_sc_gather(B, V):
    info = plsc.get_sparse_core_info()
    NC, NS, L = info.num_cores, info.num_subcores, info.num_lanes
    NW = NC * NS
    BPW = B // NW
    assert B % (8 * NW) == 0

    mesh = plsc.VectorSubcoreMesh(core_axis_name="c", subcore_axis_name="s")

    @functools.partial(
        pl.kernel,
        mesh=mesh,
        out_type=jax.ShapeDtypeStruct((B,), jnp.float32),
        scratch_types=[
            pltpu.VMEM((BPW,), jnp.int32),
            pltpu.VMEM((BPW,), jnp.float32),
            pltpu.SemaphoreType.DMA,
        ],
    )
    def sc_gather(strips_hbm, tgt_hbm, tv_hbm, idx_v, vals_v, sem):
        wid = lax.axis_index("s") * NC + lax.axis_index("c")
        base = wid * BPW
        pltpu.sync_copy(tgt_hbm.at[pl.ds(base, BPW)], idx_v)
        for cc in range(BPW // L):
            t = idx_v[pl.ds(cc * L, L)]
            cols = (base + cc * L) + lax.iota(jnp.int32, L)
            idx_v[pl.ds(cc * L, L)] = (t & 7) * B + cols
        pltpu.async_copy(strips_hbm.at[idx_v], vals_v, sem).wait()
        pltpu.sync_copy(vals_v, tv_hbm.at[pl.ds(base, BPW)])

    return sc_gather


def _count_body(B, V, BNV, C, tv_ref, tgt_ref, preds_ref, out_ref, acc_ref):
    c = pl.program_id(0)

    @pl.when(c == 0)
    def _init():
        acc_ref[...] = jnp.zeros_like(acc_ref)

    def accumulate(mask_tail):
        x = preds_ref[...]
        tv = tv_ref[...]
        ti = tgt_ref[...]
        s = lax.bitcast_convert_type(tv, jnp.int32)
        sm = jnp.where(s > 0, s - 1,
                       jnp.where(s == 0, jnp.int32(-2147483647), s + 1))
        tvm = lax.bitcast_convert_type(sm, jnp.float32)
        pos = c * BNV + lax.broadcasted_iota(jnp.int32, (BNV, B), 0)
        thresh = jnp.where(pos < ti, tvm, tv)
        better = x > thresh
        if mask_tail:
            better &= pos < V
        bf = jnp.where(better, 1.0, 0.0)
        part = bf[0:8]
        for k in range(1, BNV // 8):
            part = part + bf[8 * k:8 * k + 8]
        acc_ref[...] += part

    @pl.when(c < C - 1)
    def _mid():
        accumulate(False)

    @pl.when(c == C - 1)
    def _fin():
        accumulate(True)
        rank = jnp.sum(acc_ref[...], axis=0, keepdims=True)
        c1 = jnp.sum((rank < 0.5).astype(jnp.float32))
        c5 = jnp.sum((rank < 4.5).astype(jnp.float32))
        out_ref[0] = (1.0 - c1 / B) * 100.0
        out_ref[1] = (1.0 - c5 / B) * 100.0


def _make_tc_count(B, V, BNV=4096):
    C = pl.cdiv(V, BNV)
    return pl.pallas_call(
        functools.partial(_count_body, B, V, BNV, C),
        grid=(C,),
        in_specs=[
            pl.BlockSpec((1, B), lambda c: (0, 0)),
            pl.BlockSpec((1, B), lambda c: (0, 0)),
            pl.BlockSpec((BNV, B), lambda c: (c, 0)),
        ],
        out_specs=pl.BlockSpec(memory_space=pltpu.SMEM),
        out_shape=jax.ShapeDtypeStruct((2,), jnp.float32),
        scratch_shapes=[pltpu.VMEM((8, B), jnp.float32)],
        compiler_params=pltpu.CompilerParams(
            dimension_semantics=("arbitrary",),
        ),
    )


def kernel(preds, targets):
    B, V = preds.shape
    targets = targets.astype(jnp.int32)
    predsT = preds.T
    strips = _make_strip_stage(B, V)(targets, predsT)
    tv = _make_sc_gather(B, V)(strips.reshape(-1), targets)
    errs = _make_tc_count(B, V)(
        tv.reshape(1, B), targets.reshape(1, B), predsT
    )
    return (errs[0], errs[1])
